# baseline (speedup 1.0000x reference)
import jax
import jax.numpy as jnp
from jax import lax
from jax.experimental import pallas as pl
from jax.experimental.pallas import tpu as pltpu
from jax.experimental.pallas import tpu_sc as plsc

N_NODES = 10000
N_EDGES = 320000
D_FEAT = 128

NC = 2
NS = 16
NW = NC * NS

E_PER_W = N_EDGES // NW
CHUNK = 128
NBUF = 5
N_CHUNKS = 80
LAST_OFF = E_PER_W - CHUNK


def _gather_body(h_hbm, src_hbm, out_hbm, idx_all, row_bufs, gat_sems, out_sems):
    wid = lax.axis_index("s") * NC + lax.axis_index("c")
    base = wid * E_PER_W

    pltpu.sync_copy(src_hbm.at[pl.ds(base, E_PER_W)], idx_all)

    def chunk_off(j):
        return jnp.minimum(j * CHUNK, LAST_OFF)

    def fill(b, j):
        off = chunk_off(j)
        pltpu.async_copy(
            h_hbm.at[idx_all.at[pl.ds(off, CHUNK)]], row_bufs[b], gat_sems[b]
        )

    def wait_fill(b, j):
        off = chunk_off(j)
        pltpu.make_async_copy(
            h_hbm.at[idx_all.at[pl.ds(off, CHUNK)]], row_bufs[b], gat_sems[b]
        ).wait()

    def scatter(b, j):
        pltpu.async_copy(
            row_bufs[b], out_hbm.at[pl.ds(base + chunk_off(j), CHUNK)], out_sems[b]
        )

    def wait_scatter(b, j):
        pltpu.make_async_copy(
            row_bufs[b], out_hbm.at[pl.ds(base + chunk_off(j), CHUNK)], out_sems[b]
        ).wait()

    for b in range(NBUF):
        fill(b, b)

    def group(gi, carry):
        g = gi * NBUF
        for b in range(NBUF):
            wait_fill(b, g + b)
            scatter(b, g + b)
        for b in range(NBUF):
            wait_scatter(b, g + b)
            fill(b, g + b + NBUF)
        return carry

    lax.fori_loop(0, N_CHUNKS // NBUF - 1, group, 0)

    tail = N_CHUNKS - NBUF
    for b in range(NBUF):
        wait_fill(b, tail + b)
        scatter(b, tail + b)
    for b in range(NBUF):
        wait_scatter(b, tail + b)


def _sc_gather(h, src):
    mesh = plsc.VectorSubcoreMesh(
        core_axis_name="c", subcore_axis_name="s", num_cores=NC, num_subcores=NS
    )
    scratch = (
        pltpu.VMEM((E_PER_W,), jnp.int32),
        [pltpu.VMEM((CHUNK, D_FEAT), jnp.float32) for _ in range(NBUF)],
        [pltpu.SemaphoreType.DMA for _ in range(NBUF)],
        [pltpu.SemaphoreType.DMA for _ in range(NBUF)],
    )
    run = pl.kernel(
        _gather_body,
        out_type=jax.ShapeDtypeStruct((N_EDGES, D_FEAT), jnp.float32),
        mesh=mesh,
        scratch_types=scratch,
        name="sc_edge_gather",
    )
    return run(h, src)


@jax.jit
def kernel(h, edge_index):
    src = edge_index[0].astype(jnp.int32)
    return _sc_gather(h, src)


# speedup vs baseline: 1.1200x; 1.1200x over previous
import jax
import jax.numpy as jnp
from jax import lax
from jax.experimental import pallas as pl
from jax.experimental.pallas import tpu as pltpu
from jax.experimental.pallas import tpu_sc as plsc

N_NODES = 10000
N_EDGES = 320000
D_FEAT = 128

NC = 2
NS = 16
NW = NC * NS

E_PER_W = N_EDGES // NW
CHUNK = 128
NBUF = 2
N_CHUNKS = 80
LAST_OFF = E_PER_W - CHUNK


ROWS_PER_TILE = 624


def _gather_body(h_hbm, src_hbm, out_hbm, h_spmem, idx_all, row_bufs,
                 gat_sems, out_sems):
    cid = lax.axis_index("c")
    sid = lax.axis_index("s")
    wid = sid * NC + cid
    base = wid * E_PER_W

    stage_off = sid * ROWS_PER_TILE
    stage_len = jnp.where(sid == NS - 1, N_NODES - (NS - 1) * ROWS_PER_TILE,
                          ROWS_PER_TILE)
    pltpu.sync_copy(h_hbm.at[pl.ds(stage_off, ROWS_PER_TILE)],
                    h_spmem.at[pl.ds(stage_off, ROWS_PER_TILE)])
    del stage_len
    rem_off = NS * ROWS_PER_TILE
    rem = N_NODES - rem_off

    @pl.when(sid == NS - 1)
    def _stage_rem():
        pltpu.sync_copy(h_hbm.at[pl.ds(rem_off, rem)],
                        h_spmem.at[pl.ds(rem_off, rem)])

    pltpu.sync_copy(src_hbm.at[pl.ds(base, E_PER_W)], idx_all)
    plsc.subcore_barrier()

    def chunk_off(j):
        return jnp.minimum(j * CHUNK, LAST_OFF)

    def fill(b, j):
        off = chunk_off(j)
        pltpu.async_copy(
            h_spmem.at[idx_all.at[pl.ds(off, CHUNK)]], row_bufs[b], gat_sems[b]
        )

    def wait_fill(b, j):
        off = chunk_off(j)
        pltpu.make_async_copy(
            h_spmem.at[idx_all.at[pl.ds(off, CHUNK)]], row_bufs[b], gat_sems[b]
        ).wait()

    def scatter(b, j):
        pltpu.async_copy(
            row_bufs[b], out_hbm.at[pl.ds(base + chunk_off(j), CHUNK)], out_sems[b]
        )

    def wait_scatter(b, j):
        pltpu.make_async_copy(
            row_bufs[b], out_hbm.at[pl.ds(base + chunk_off(j), CHUNK)], out_sems[b]
        ).wait()

    for b in range(NBUF):
        fill(b, b)

    def group(gi, carry):
        g = gi * NBUF
        for b in range(NBUF):
            wait_fill(b, g + b)
            scatter(b, g + b)
        for b in range(NBUF):
            wait_scatter(b, g + b)
            fill(b, g + b + NBUF)
        return carry

    lax.fori_loop(0, N_CHUNKS // NBUF - 1, group, 0)

    tail = N_CHUNKS - NBUF
    for b in range(NBUF):
        wait_fill(b, tail + b)
        scatter(b, tail + b)
    for b in range(NBUF):
        wait_scatter(b, tail + b)


def _sc_gather(h, src):
    mesh = plsc.VectorSubcoreMesh(
        core_axis_name="c", subcore_axis_name="s", num_cores=NC, num_subcores=NS
    )
    scratch = (
        pltpu.VMEM_SHARED((N_NODES, D_FEAT), jnp.float32),
        pltpu.VMEM((E_PER_W,), jnp.int32),
        [pltpu.VMEM((CHUNK, D_FEAT), jnp.float32) for _ in range(NBUF)],
        [pltpu.SemaphoreType.DMA for _ in range(NBUF)],
        [pltpu.SemaphoreType.DMA for _ in range(NBUF)],
    )
    run = pl.kernel(
        _gather_body,
        out_type=jax.ShapeDtypeStruct((N_EDGES, D_FEAT), jnp.float32),
        mesh=mesh,
        scratch_types=scratch,
        name="sc_edge_gather",
    )
    return run(h, src)


@jax.jit
def kernel(h, edge_index):
    src = edge_index[0].astype(jnp.int32)
    return _sc_gather(h, src)


# speedup vs baseline: 1.3273x; 1.1851x over previous
import jax
import jax.numpy as jnp
from jax import lax
from jax.experimental import pallas as pl
from jax.experimental.pallas import tpu as pltpu
from jax.experimental.pallas import tpu_sc as plsc

N_NODES = 10000
N_EDGES = 320000
D_FEAT = 128

NC = 2
NS = 16
NW = NC * NS

E_PER_W = N_EDGES // NW
CHUNK = 128
NBUF = 3
N_CHUNKS = 80
LAST_OFF = E_PER_W - CHUNK


ROWS_PER_TILE = 624


def _gather_body(h_hbm, src_hbm, out_hbm, h_spmem, idx_bufs, row_bufs,
                 gat_sems, out_sems):
    cid = lax.axis_index("c")
    sid = lax.axis_index("s")
    wid = sid * NC + cid
    base = wid * E_PER_W

    stage_off = sid * ROWS_PER_TILE
    stage_len = jnp.where(sid == NS - 1, N_NODES - (NS - 1) * ROWS_PER_TILE,
                          ROWS_PER_TILE)
    pltpu.sync_copy(h_hbm.at[pl.ds(stage_off, ROWS_PER_TILE)],
                    h_spmem.at[pl.ds(stage_off, ROWS_PER_TILE)])
    del stage_len
    rem_off = NS * ROWS_PER_TILE
    rem = N_NODES - rem_off

    @pl.when(sid == NS - 1)
    def _stage_rem():
        pltpu.sync_copy(h_hbm.at[pl.ds(rem_off, rem)],
                        h_spmem.at[pl.ds(rem_off, rem)])

    plsc.subcore_barrier()

    def chunk_off(j):
        return jnp.minimum(j * CHUNK, LAST_OFF)

    def fill(b, j):
        off = chunk_off(j)
        pltpu.sync_copy(src_hbm.at[pl.ds(base + off, CHUNK)], idx_bufs[b])
        pltpu.async_copy(
            h_spmem.at[idx_bufs[b]], row_bufs[b], gat_sems[b]
        )

    def wait_fill(b, j):
        pltpu.make_async_copy(
            h_spmem.at[idx_bufs[b]], row_bufs[b], gat_sems[b]
        ).wait()

    def scatter(b, j):
        pltpu.async_copy(
            row_bufs[b], out_hbm.at[pl.ds(base + chunk_off(j), CHUNK)], out_sems[b]
        )

    def wait_scatter(b, j):
        pltpu.make_async_copy(
            row_bufs[b], out_hbm.at[pl.ds(base + chunk_off(j), CHUNK)], out_sems[b]
        ).wait()

    for b in range(NBUF):
        fill(b, b)

    def group(gi, carry):
        g = gi * NBUF
        for b in range(NBUF):
            wait_fill(b, g + b)
            scatter(b, g + b)
        for b in range(NBUF):
            wait_scatter(b, g + b)
            fill(b, g + b + NBUF)
        return carry

    lax.fori_loop(0, N_CHUNKS // NBUF - 1, group, 0)

    tail = N_CHUNKS - NBUF
    for b in range(NBUF):
        wait_fill(b, tail + b)
        scatter(b, tail + b)
    for b in range(NBUF):
        wait_scatter(b, tail + b)


def _sc_gather(h, src):
    mesh = plsc.VectorSubcoreMesh(
        core_axis_name="c", subcore_axis_name="s", num_cores=NC, num_subcores=NS
    )
    scratch = (
        pltpu.VMEM_SHARED((N_NODES, D_FEAT), jnp.float32),
        [pltpu.VMEM((CHUNK,), jnp.int32) for _ in range(NBUF)],
        [pltpu.VMEM((CHUNK, D_FEAT), jnp.float32) for _ in range(NBUF)],
        [pltpu.SemaphoreType.DMA for _ in range(NBUF)],
        [pltpu.SemaphoreType.DMA for _ in range(NBUF)],
    )
    run = pl.kernel(
        _gather_body,
        out_type=jax.ShapeDtypeStruct((N_EDGES, D_FEAT), jnp.float32),
        mesh=mesh,
        scratch_types=scratch,
        name="sc_edge_gather",
    )
    return run(h, src)


@jax.jit
def kernel(h, edge_index):
    src = edge_index[0].astype(jnp.int32)
    return _sc_gather(h, src)
